# baseline (speedup 1.0000x reference)
import functools

import jax
import jax.numpy as jnp
from jax.experimental import pallas as pl
from jax.experimental.pallas import tpu as pltpu

B = 256
IN_DIM = 512
OUT_DIM = 10000
BN = 256


def _gcn_body(x_ref, w_ref, adj_ref, bias_ref, out_ref, support_ref):
    @pl.when(pl.program_id(0) == 0)
    def _compute_support():
        s = jnp.dot(
            x_ref[...].astype(jnp.bfloat16),
            w_ref[...].astype(jnp.bfloat16),
            preferred_element_type=jnp.float32,
        )
        support_ref[...] = s.astype(jnp.bfloat16)

    a = adj_ref[...].astype(jnp.bfloat16)
    acc = jax.lax.dot_general(
        support_ref[...],
        a,
        dimension_numbers=(((1,), (1,)), ((), ())),
        preferred_element_type=jnp.float32,
    )
    out_ref[...] = acc + bias_ref[...]


@functools.partial(jax.jit, static_argnames=())
def kernel(input, adj, weight, bias):
    bias2d = bias.reshape(1, OUT_DIM)
    grid = (pl.cdiv(OUT_DIM, BN),)
    out = pl.pallas_call(
        _gcn_body,
        grid=grid,
        in_specs=[
            pl.BlockSpec((B, IN_DIM), lambda n: (0, 0)),
            pl.BlockSpec((IN_DIM, OUT_DIM), lambda n: (0, 0)),
            pl.BlockSpec((BN, OUT_DIM), lambda n: (n, 0)),
            pl.BlockSpec((1, BN), lambda n: (0, n)),
        ],
        out_specs=pl.BlockSpec((B, BN), lambda n: (0, n)),
        out_shape=jax.ShapeDtypeStruct((B, OUT_DIM), jnp.float32),
        scratch_shapes=[pltpu.VMEM((B, OUT_DIM), jnp.bfloat16)],
        compiler_params=pltpu.CompilerParams(
            dimension_semantics=("arbitrary",),
        ),
    )(input, weight, adj, bias2d)
    return out
